# baseline (speedup 1.0000x reference)
import functools
import jax
import jax.numpy as jnp
from jax import lax
from jax.experimental import pallas as pl
from jax.experimental.pallas import tpu as pltpu
from jax.experimental.pallas import tpu_sc as plsc

IN_F = 784
OUT_F = 128
NK = 20
NB = 512
NWORK = 32
RPW = NB // NWORK
NCH = IN_F // 16
UNR = 4

_FMAX = float(jnp.finfo(jnp.float32).max)
_FMIN = float(jnp.finfo(jnp.float32).min)


def _first(x):
    return x[0] if isinstance(x, (tuple, list)) else x


def _sortd(c):
    return _first(plsc.sort_key_val(c, c, descending=True))


def _sorta(c):
    return _first(plsc.sort_key_val(c, c))


def _merge(st, c):
    A, B, A2, B2 = st
    sd = _sortd(c)
    sa = lax.rev(sd, (0,))
    hi = jnp.maximum(A, sa)
    lo = jnp.minimum(A, sa)
    A = _sortd(hi)
    los = _sorta(lo)
    B = _sortd(jnp.maximum(B, los))
    lo2 = jnp.minimum(A2, sd)
    hi2 = jnp.maximum(A2, sd)
    A2 = _sorta(lo2)
    his = _sortd(hi2)
    B2 = _sorta(jnp.minimum(B2, his))
    return A, B, A2, B2


def _sc_corr(x_flat, w_flat):
    mesh = plsc.VectorSubcoreMesh(core_axis_name="c", subcore_axis_name="s")

    @functools.partial(
        pl.kernel, mesh=mesh,
        compiler_params=pltpu.CompilerParams(needs_layout_passes=False),
        out_type=jax.ShapeDtypeStruct((NB * OUT_F,), jnp.float32),
        scratch_types=[
            pltpu.VMEM((RPW * IN_F,), jnp.float32),
            pltpu.VMEM((OUT_F * IN_F,), jnp.float32),
            pltpu.VMEM((RPW * OUT_F,), jnp.float32),
        ],
    )
    def sck(x_hbm, w_hbm, out_hbm, xv, wv, cv):
        wid = lax.axis_index("s") * 2 + lax.axis_index("c")
        pltpu.sync_copy(x_hbm.at[pl.ds(wid * (RPW * IN_F), RPW * IN_F)], xv)
        pltpu.sync_copy(w_hbm, wv)

        lanes = lax.iota(jnp.int32, 16)
        four = jnp.int32(4)
        zero = jnp.zeros((16,), jnp.float32)

        def o_body(og, r):
            acc = zero
            for g in range(16 // UNR):
                o0 = og * 16 + g * UNR

                def j_body(j, st, o0=o0):
                    xs = xv[pl.ds(r * IN_F + j * 16, 16)]
                    out = []
                    for u in range(UNR):
                        ws = wv[pl.ds((o0 + u) * IN_F + j * 16, 16)]
                        out.append(_merge(st[u], xs * ws))
                    return tuple(out)

                init1 = (jnp.full((16,), _FMIN, jnp.float32),
                         jnp.full((16,), _FMIN, jnp.float32),
                         jnp.full((16,), _FMAX, jnp.float32),
                         jnp.full((16,), _FMAX, jnp.float32))
                st = lax.fori_loop(0, NCH, j_body, (init1,) * UNR)

                for u in range(UNR):
                    A, B, A2, B2 = st[u]
                    top20 = jnp.sum(A) + jnp.sum(jnp.where(lanes < four, B, zero))
                    bot20 = jnp.sum(A2) + jnp.sum(jnp.where(lanes < four, B2, zero))
                    acc = jnp.where(lanes == jnp.int32(g * UNR + u),
                                    -(top20 + bot20), acc)
            cv[pl.ds(r * OUT_F + og * 16, 16)] = acc
            return r

        def r_body(r, _):
            lax.fori_loop(0, OUT_F // 16, o_body, r)
            return 0

        lax.fori_loop(0, RPW, r_body, 0)
        pltpu.sync_copy(cv, out_hbm.at[pl.ds(wid * (RPW * OUT_F), RPW * OUT_F)])

    return sck(x_flat, w_flat)


def _tc_body(x_ref, w_ref, b_ref, c_ref, o_ref):
    dot = jax.lax.dot_general(
        x_ref[...], w_ref[...], dimension_numbers=(((1,), (1,)), ((), ())),
        preferred_element_type=jnp.float32)
    o_ref[...] = dot + c_ref[...] + b_ref[...]


def kernel(x, W, b):
    corr = _sc_corr(x.reshape(-1), W.reshape(-1)).reshape(NB, OUT_F)
    b2 = b.reshape(1, OUT_F)
    return pl.pallas_call(
        _tc_body,
        out_shape=jax.ShapeDtypeStruct((NB, OUT_F), jnp.float32),
    )(x, W, b2, corr)


# speedup vs baseline: 1.3608x; 1.3608x over previous
import functools
import jax
import jax.numpy as jnp
from jax import lax
from jax.experimental import pallas as pl
from jax.experimental.pallas import tpu as pltpu
from jax.experimental.pallas import tpu_sc as plsc

IN_F = 784
OUT_F = 128
NK = 20
NB = 512
NWORK = 32
RPW = NB // NWORK
NCH = IN_F // 16
UNR = 4

_FMAX = float(jnp.finfo(jnp.float32).max)
_FMIN = float(jnp.finfo(jnp.float32).min)


def _first(x):
    return x[0] if isinstance(x, (tuple, list)) else x


def _sortd(c):
    return _first(plsc.sort_key_val(c, c, descending=True))


def _sorta(c):
    return _first(plsc.sort_key_val(c, c))


def _merge1(st, c):
    A, B, A2, B2 = st
    sa = _sorta(c)
    hi = jnp.maximum(A, sa)
    lo = jnp.minimum(A, sa)
    A = _sortd(hi)
    B = _sortd(jnp.maximum(B, _sorta(lo)))
    lo2 = jnp.minimum(A2, sa)
    hi2 = jnp.maximum(A2, sa)
    A2 = _sortd(lo2)
    B2 = _sortd(jnp.minimum(B2, _sorta(hi2)))
    return A, B, A2, B2


def _merge2(st, c1, c2):
    A, B, A2, B2 = st
    sa1 = _sorta(c1)
    sd2 = _sortd(c2)
    hi = jnp.maximum(sa1, sd2)
    lo = jnp.minimum(sa1, sd2)
    Ha = _sorta(hi)
    La = _sorta(lo)
    u1 = jnp.maximum(A, La)
    u2 = jnp.maximum(B, Ha)
    A = _sortd(jnp.maximum(u1, u2))
    B = _sortd(jnp.minimum(u1, u2))
    w1 = jnp.minimum(B2, La)
    w2 = jnp.minimum(A2, Ha)
    B2 = _sortd(jnp.maximum(w1, w2))
    A2 = _sortd(jnp.minimum(w1, w2))
    return A, B, A2, B2


def _sc_corr(x_flat, w_flat):
    mesh = plsc.VectorSubcoreMesh(core_axis_name="c", subcore_axis_name="s")

    @functools.partial(
        pl.kernel, mesh=mesh,
        compiler_params=pltpu.CompilerParams(needs_layout_passes=False),
        out_type=jax.ShapeDtypeStruct((NB * OUT_F,), jnp.float32),
        scratch_types=[
            pltpu.VMEM((RPW * IN_F,), jnp.float32),
            pltpu.VMEM((OUT_F * IN_F,), jnp.float32),
            pltpu.VMEM((RPW * OUT_F,), jnp.float32),
        ],
    )
    def sck(x_hbm, w_hbm, out_hbm, xv, wv, cv):
        wid = lax.axis_index("s") * 2 + lax.axis_index("c")
        pltpu.sync_copy(x_hbm.at[pl.ds(wid * (RPW * IN_F), RPW * IN_F)], xv)
        pltpu.sync_copy(w_hbm, wv)

        lanes = lax.iota(jnp.int32, 16)
        four = jnp.int32(4)
        zero = jnp.zeros((16,), jnp.float32)

        def o_body(og, r):
            acc = zero
            for g in range(16 // UNR):
                o0 = og * 16 + g * UNR

                def j_body(j, st, o0=o0):
                    xs1 = xv[pl.ds(r * IN_F + j * 32, 16)]
                    xs2 = xv[pl.ds(r * IN_F + j * 32 + 16, 16)]
                    out = []
                    for u in range(UNR):
                        ws1 = wv[pl.ds((o0 + u) * IN_F + j * 32, 16)]
                        ws2 = wv[pl.ds((o0 + u) * IN_F + j * 32 + 16, 16)]
                        out.append(_merge2(st[u], xs1 * ws1, xs2 * ws2))
                    return tuple(out)

                init1 = (jnp.full((16,), _FMIN, jnp.float32),
                         jnp.full((16,), _FMIN, jnp.float32),
                         jnp.full((16,), _FMAX, jnp.float32),
                         jnp.full((16,), _FMAX, jnp.float32))
                st = lax.fori_loop(0, NCH // 2, j_body, (init1,) * UNR)
                xst = xv[pl.ds(r * IN_F + (NCH - 1) * 16, 16)]
                st = tuple(
                    _merge1(st[u],
                            xst * wv[pl.ds((o0 + u) * IN_F + (NCH - 1) * 16, 16)])
                    for u in range(UNR))

                for u in range(UNR):
                    A, B, A2, B2 = st[u]
                    top20 = jnp.sum(A) + jnp.sum(jnp.where(lanes < four, B, zero))
                    bot20 = jnp.sum(A2) + jnp.sum(jnp.where(lanes >= jnp.int32(12), B2, zero))
                    acc = jnp.where(lanes == jnp.int32(g * UNR + u),
                                    -(top20 + bot20), acc)
            cv[pl.ds(r * OUT_F + og * 16, 16)] = acc
            return r

        def r_body(r, _):
            lax.fori_loop(0, OUT_F // 16, o_body, r)
            return 0

        lax.fori_loop(0, RPW, r_body, 0)
        pltpu.sync_copy(cv, out_hbm.at[pl.ds(wid * (RPW * OUT_F), RPW * OUT_F)])

    return sck(x_flat, w_flat)


def _tc_body(x_ref, w_ref, b_ref, c_ref, o_ref):
    dot = jax.lax.dot_general(
        x_ref[...], w_ref[...], dimension_numbers=(((1,), (1,)), ((), ())),
        preferred_element_type=jnp.float32)
    o_ref[...] = dot + c_ref[...] + b_ref[...]


def kernel(x, W, b):
    corr = _sc_corr(x.reshape(-1), W.reshape(-1)).reshape(NB, OUT_F)
    b2 = b.reshape(1, OUT_F)
    return pl.pallas_call(
        _tc_body,
        out_shape=jax.ShapeDtypeStruct((NB, OUT_F), jnp.float32),
    )(x, W, b2, corr)
